# baseline (speedup 1.0000x reference)
import dataclasses

import jax
import jax.numpy as jnp
from jax import lax
from jax.experimental import pallas as pl
from jax.experimental.pallas import tpu as pltpu
from jax.experimental.pallas import tpu_sc as plsc

N = 10000
E = 320000
EP = 327680
IDX_ROWS = EP // 128
ACC_N = 10112
TRASH = 10016
DHID = 256
DOUT = 128
TCOLS = 144
RB = 2000
GRID = N // RB


def _mesh():
    return plsc.VectorSubcoreMesh(core_axis_name="c", subcore_axis_name="s")


def _sc_params():
    cp = pltpu.CompilerParams()
    if "needs_layout_passes" in pltpu.CompilerParams.__dataclass_fields__:
        cp = dataclasses.replace(cp, needs_layout_passes=False)
    return cp


def _sc_deg_body(dst_hbm, deg_hbm, dstv, onesv, zerov, acc):
    c = lax.axis_index("c")
    s = lax.axis_index("s")

    @pl.loop(0, 128)
    def _(i):
        onesv[i, :] = jnp.ones((16,), jnp.float32)

    @pl.loop(0, 8)
    def _(i):
        zerov[i, :] = jnp.zeros((16,), jnp.float32)

    @pl.loop(0, 79)
    def _(k):
        pltpu.sync_copy(zerov, acc.at[pl.ds(s * 632 + k * 8, 8)])

    plsc.subcore_barrier()

    base = c * 1280 + s * 80
    pltpu.sync_copy(dst_hbm.at[pl.ds(base, 80)], dstv)

    @pl.loop(0, 80)
    def _(j):
        pltpu.sync_copy(onesv, acc.at[dstv.at[j]], add=True)

    plsc.subcore_barrier()
    pltpu.sync_copy(acc.at[pl.ds(s * 632, 632)], deg_hbm.at[c, pl.ds(s * 632, 632)])


def _sc_deg(dst_p):
    kern = pl.kernel(
        _sc_deg_body,
        out_type=jax.ShapeDtypeStruct((2, ACC_N, 16), jnp.float32),
        mesh=_mesh(),
        scratch_types=[
            pltpu.VMEM((80, 128), jnp.int32),
            pltpu.VMEM((128, 16), jnp.float32),
            pltpu.VMEM((8, 16), jnp.float32),
            pltpu.VMEM_SHARED((ACC_N, 16), jnp.float32),
        ],
    )
    return kern(dst_p)


def _sc_gcn_body(g_hbm, src_hbm, dst_hbm, out_hbm,
                 srcv, dstv, r0, r1, acc, g0, g1):
    c = lax.axis_index("c")
    s = lax.axis_index("s")

    @pl.loop(0, 8)
    def _(i):
        for k in range(8):
            r0[i, pl.ds(k * 16, 16)] = jnp.zeros((16,), jnp.float32)

    @pl.loop(0, 79)
    def _(k):
        pltpu.sync_copy(r0.at[pl.ds(0, 8)], acc.at[pl.ds(s * 632 + k * 8, 8)])

    plsc.subcore_barrier()

    off = jnp.full((16,), c * N, jnp.int32)
    base = s * 320

    @pl.loop(0, 40)
    def _(t):
        pltpu.sync_copy(src_hbm.at[pl.ds(base + t * 8, 8)], srcv)
        pltpu.sync_copy(dst_hbm.at[pl.ds(base + t * 8, 8)], dstv)

        @pl.loop(0, 8)
        def _(j):
            for k in range(4):
                sl = pl.ds(k * 16, 16)
                srcv[j, sl] = srcv[j, sl] + off

        for p in range(4):
            cp0 = pltpu.async_copy(g_hbm.at[srcv.at[2 * p]], r0, g0)
            cp1 = pltpu.async_copy(g_hbm.at[srcv.at[2 * p + 1]], r1, g1)
            cp0.wait()
            pltpu.sync_copy(r0, acc.at[dstv.at[2 * p]], add=True)
            cp1.wait()
            pltpu.sync_copy(r1, acc.at[dstv.at[2 * p + 1]], add=True)

    plsc.subcore_barrier()
    pltpu.sync_copy(acc.at[pl.ds(s * 632, 632)], out_hbm.at[c, pl.ds(s * 632, 632)])


def _sc_gcn(g_tab, src_p64, dst_p64):
    kern = pl.kernel(
        _sc_gcn_body,
        out_type=jax.ShapeDtypeStruct((2, ACC_N, 128), jnp.float32),
        mesh=_mesh(),
        scratch_types=[
            pltpu.VMEM((8, 64), jnp.int32),
            pltpu.VMEM((8, 64), jnp.int32),
            pltpu.VMEM((64, 128), jnp.float32),
            pltpu.VMEM((64, 128), jnp.float32),
            pltpu.VMEM_SHARED((ACC_N, 128), jnp.float32),
            pltpu.SemaphoreType.DMA,
            pltpu.SemaphoreType.DMA,
        ],
    )
    return kern(g_tab, src_p64, dst_p64)


def _sc_class_body(src_hbm, dst_hbm, as_hbm, ad_hbm, ca_hbm,
                   gidx_hbm, dstc_hbm, den_hbm,
                   sv, dv, gv, d0v, d1v, asv, adv, cavv, denp0, denp1):
    c = lax.axis_index("c")
    s = lax.axis_index("s")

    for t in range(7):
        adv[pl.ds(N + t * 16, 16)] = jnp.zeros((16,), jnp.float32)
    pltpu.sync_copy(as_hbm, asv)
    pltpu.sync_copy(ad_hbm, adv.at[pl.ds(0, N)])
    pltpu.sync_copy(ca_hbm, cavv)

    @pl.loop(0, 632)
    def _(i):
        denp0[pl.ds(i * 16, 16)] = jnp.zeros((16,), jnp.float32)
        denp1[pl.ds(i * 16, 16)] = jnp.zeros((16,), jnp.float32)

    cavec = cavv[...]
    trash16 = jnp.full((16,), TRASH, jnp.int32)
    n16 = jnp.full((16,), N, jnp.int32)
    base = c * 1280 + s * 80

    @pl.loop(0, 10)
    def _(t):
        pltpu.sync_copy(src_hbm.at[pl.ds(base + t * 8, 8)], sv)
        pltpu.sync_copy(dst_hbm.at[pl.ds(base + t * 8, 8)], dv)

        @pl.loop(0, 8)
        def _(j):
            for k in range(8):
                sl = pl.ds(k * 16, 16)
                s16 = sv[j, sl]
                d16 = dv[j, sl]
                av = plsc.load_gather(asv, [s16])
                advv = plsc.load_gather(adv, [d16])
                z = av + advv
                pos = z > 0.0
                d0 = jnp.where(pos, d16, trash16)
                d1 = jnp.where(pos, trash16, d16)
                w16 = jnp.exp(jnp.where(pos, av, 0.2 * av) - cavec)
                gv[j, sl] = jnp.where(pos, s16, s16 + n16)
                d0v[j, sl] = d0
                d1v[j, sl] = d1
                plsc.addupdate_scatter(denp0, [d0], w16)
                plsc.addupdate_scatter(denp1, [d1], w16)

        pltpu.sync_copy(gv, gidx_hbm.at[pl.ds(base + t * 8, 8)])
        pltpu.sync_copy(d0v, dstc_hbm.at[0, pl.ds(base + t * 8, 8)])
        pltpu.sync_copy(d1v, dstc_hbm.at[1, pl.ds(base + t * 8, 8)])

    wid = c * 16 + s
    pltpu.sync_copy(denp0, den_hbm.at[0, wid])
    pltpu.sync_copy(denp1, den_hbm.at[1, wid])


def _sc_class(src_p, dst_p, a_s, a_d, ca16):
    kern = pl.kernel(
        _sc_class_body,
        out_type=[jax.ShapeDtypeStruct((IDX_ROWS, 128), jnp.int32),
                  jax.ShapeDtypeStruct((2, IDX_ROWS, 128), jnp.int32),
                  jax.ShapeDtypeStruct((2, 32, ACC_N), jnp.float32)],
        mesh=_mesh(),
        compiler_params=_sc_params(),
        scratch_types=[
            pltpu.VMEM((8, 128), jnp.int32),
            pltpu.VMEM((8, 128), jnp.int32),
            pltpu.VMEM((8, 128), jnp.int32),
            pltpu.VMEM((8, 128), jnp.int32),
            pltpu.VMEM((8, 128), jnp.int32),
            pltpu.VMEM((N,), jnp.float32),
            pltpu.VMEM((ACC_N,), jnp.float32),
            pltpu.VMEM((16,), jnp.float32),
            pltpu.VMEM((ACC_N,), jnp.float32),
            pltpu.VMEM((ACC_N,), jnp.float32),
        ],
    )
    return kern(src_p, dst_p, a_s, a_d, ca16)


def _sc_gat_body(t_hbm, gidx_hbm, dstc_hbm, out_hbm,
                 gvv, dvv, rows0, rows1, acc, sem0, sem1):
    c = lax.axis_index("c")
    s = lax.axis_index("s")

    @pl.loop(0, 8)
    def _(i):
        for k in range(8):
            rows0[i, pl.ds(k * 16, 16)] = jnp.zeros((16,), jnp.float32)

    @pl.loop(0, 79)
    def _(k):
        pltpu.sync_copy(rows0.at[pl.ds(0, 8)], acc.at[pl.ds(s * 632 + k * 8, 8)])

    plsc.subcore_barrier()

    base = s * 160

    @pl.loop(0, 20)
    def _(t):
        pltpu.sync_copy(gidx_hbm.at[pl.ds(base + t * 8, 8)], gvv)
        pltpu.sync_copy(dstc_hbm.at[c, pl.ds(base + t * 8, 8)], dvv)

        for p in range(4):
            cp0 = pltpu.make_async_copy(t_hbm.at[gvv.at[2 * p]], rows0, sem0)
            cp1 = pltpu.make_async_copy(t_hbm.at[gvv.at[2 * p + 1]], rows1, sem1)
            cp0.start()
            cp1.start()
            cp0.wait()
            pltpu.sync_copy(rows0, acc.at[dvv.at[2 * p]], add=True)
            cp1.wait()
            pltpu.sync_copy(rows1, acc.at[dvv.at[2 * p + 1]], add=True)

    plsc.subcore_barrier()
    pltpu.sync_copy(acc.at[pl.ds(s * 632, 632)], out_hbm.at[c, pl.ds(s * 632, 632)])


def _sc_gat(t_tab, gidx, dstc):
    kern = pl.kernel(
        _sc_gat_body,
        out_type=jax.ShapeDtypeStruct((2, ACC_N, 128), jnp.float32),
        mesh=_mesh(),
        scratch_types=[
            pltpu.VMEM((8, 128), jnp.int32),
            pltpu.VMEM((8, 128), jnp.int32),
            pltpu.VMEM((128, 128), jnp.float32),
            pltpu.VMEM((128, 128), jnp.float32),
            pltpu.VMEM_SHARED((ACC_N, 128), jnp.float32),
            pltpu.SemaphoreType.DMA,
            pltpu.SemaphoreType.DMA,
        ],
    )
    return kern(t_tab, gidx, dstc)


def _mm_body(x_ref, w_ref, o_ref):
    o_ref[...] = lax.dot_general(
        x_ref[...], w_ref[...], (((1,), (0,)), ((), ())),
        precision=lax.Precision.HIGHEST)


def _tc_h(x, w_gcn):
    return pl.pallas_call(
        _mm_body,
        grid=(GRID,),
        in_specs=[pl.BlockSpec((RB, 128), lambda i: (i, 0)),
                  pl.BlockSpec((128, DHID), lambda i: (0, 0))],
        out_specs=pl.BlockSpec((RB, DHID), lambda i: (i, 0)),
        out_shape=jax.ShapeDtypeStruct((N, DHID), jnp.float32),
    )(x, w_gcn)


def _scale_body(deg_ref, h_ref, g_ref):
    deg = deg_ref[0, :, 0] + deg_ref[1, :, 0] + 1.0
    dinv = lax.rsqrt(deg)
    hb = h_ref[...]
    g_ref[0, :, :] = hb[:, :128] * dinv[:, None]
    g_ref[1, :, :] = hb[:, 128:] * dinv[:, None]


def _tc_scale(deg_raw, h):
    return pl.pallas_call(
        _scale_body,
        grid=(GRID,),
        in_specs=[pl.BlockSpec((2, RB, 16), lambda i: (0, i, 0)),
                  pl.BlockSpec((RB, DHID), lambda i: (i, 0))],
        out_specs=pl.BlockSpec((2, RB, 128), lambda i: (0, i, 0)),
        out_shape=jax.ShapeDtypeStruct((2, N, 128), jnp.float32),
    )(deg_raw, h)


def _gcn_fin_body(agg_ref, g_ref, deg_ref, bgcn_ref, wgat_ref, h2_ref):
    D = deg_ref[...]
    deg = D[0, :, 0] + D[1, :, 0] + 1.0
    dinv = lax.rsqrt(deg)
    A = agg_ref[...]
    G = g_ref[...]
    b = bgcn_ref[...]
    w = wgat_ref[...]
    hg0 = jnp.maximum((A[0] + G[0]) * dinv[:, None] + b[None, :128], 0.0)
    hg1 = jnp.maximum((A[1] + G[1]) * dinv[:, None] + b[None, 128:], 0.0)
    h2_ref[...] = (
        lax.dot_general(hg0, w[:128, :], (((1,), (0,)), ((), ())),
                        precision=lax.Precision.HIGHEST)
        + lax.dot_general(hg1, w[128:, :], (((1,), (0,)), ((), ())),
                          precision=lax.Precision.HIGHEST))


def _tc_gcn_finish(agg, gsplit2, deg_raw, b_gcn, w_gat):
    return pl.pallas_call(
        _gcn_fin_body,
        grid=(GRID,),
        in_specs=[pl.BlockSpec((2, RB, 128), lambda i: (0, i, 0)),
                  pl.BlockSpec((2, RB, 128), lambda i: (0, i, 0)),
                  pl.BlockSpec((2, RB, 16), lambda i: (0, i, 0)),
                  pl.BlockSpec((DHID,), lambda i: (0,)),
                  pl.BlockSpec((DHID, DOUT), lambda i: (0, 0))],
        out_specs=pl.BlockSpec((RB, DOUT), lambda i: (i, 0)),
        out_shape=jax.ShapeDtypeStruct((N, DOUT), jnp.float32),
    )(agg, gsplit2, deg_raw, b_gcn, w_gat)


def _att_body(h2_ref, asrc_ref, adst_ref,
              as_ref, ad_ref, fs_ref, aux_ref, ca16_ref):
    h2 = h2_ref[...]
    a_s = jnp.sum(h2 * asrc_ref[...][None, :], axis=1)
    a_d = jnp.sum(h2 * adst_ref[...][None, :], axis=1)
    ca = jnp.max(a_s)
    cd = jnp.max(a_d)
    f = jnp.exp(a_s - ca)
    f2 = jnp.exp(0.2 * a_s - ca)
    gd = jnp.exp(a_d - cd)
    g2d = jnp.exp(0.2 * a_d - cd)
    zs = a_s + a_d
    ws = jnp.exp(jnp.where(zs > 0.0, zs, 0.2 * zs) - ca - cd)
    zpad = jnp.zeros((N, 14), jnp.float32)
    as_ref[...] = a_s
    ad_ref[...] = a_d
    fs_ref[...] = jnp.concatenate([f[:, None], f2[:, None], zpad], axis=1)
    aux_ref[...] = jnp.concatenate(
        [gd[:, None], g2d[:, None], ws[:, None], zpad[:, :13]], axis=1)
    ca16_ref[...] = jnp.full((16,), ca, jnp.float32)


def _tc_att(h2, att_src, att_dst):
    return pl.pallas_call(
        _att_body,
        out_shape=[jax.ShapeDtypeStruct((N,), jnp.float32),
                   jax.ShapeDtypeStruct((N,), jnp.float32),
                   jax.ShapeDtypeStruct((N, 16), jnp.float32),
                   jax.ShapeDtypeStruct((N, 16), jnp.float32),
                   jax.ShapeDtypeStruct((16,), jnp.float32)],
    )(h2, att_src, att_dst)


def _build_t_body(h2_ref, fs_ref, t_ref):
    h2 = h2_ref[...]
    fs = fs_ref[...]
    t_ref[0, :, :] = fs[:, 0][:, None] * h2
    t_ref[1, :, :] = fs[:, 1][:, None] * h2


def _tc_build_t(h2, fs16):
    return pl.pallas_call(
        _build_t_body,
        grid=(GRID,),
        in_specs=[pl.BlockSpec((RB, DOUT), lambda i: (i, 0)),
                  pl.BlockSpec((RB, 16), lambda i: (i, 0))],
        out_specs=pl.BlockSpec((2, RB, DOUT), lambda i: (0, i, 0)),
        out_shape=jax.ShapeDtypeStruct((2, N, DOUT), jnp.float32),
    )(h2, fs16)


def _final_body(acc_ref, den_ref, aux_ref, h2_ref, bgat_ref, o_ref):
    A = acc_ref[...]
    dn = den_ref[...]
    aux = aux_ref[...]
    h2 = h2_ref[...]
    gd = aux[:, 0]
    g2d = aux[:, 1]
    ws = aux[:, 2]
    num = (A[0, :, :] * gd[:, None] + A[1, :, :] * g2d[:, None]
           + h2 * ws[:, None])
    den = jnp.sum(dn[0], axis=1) * gd + jnp.sum(dn[1], axis=1) * g2d + ws
    o_ref[...] = num / den[:, None] + bgat_ref[...][None, :]


def _tc_final(accg, den_raw, aux, h2, b_gat):
    return pl.pallas_call(
        _final_body,
        grid=(GRID,),
        in_specs=[pl.BlockSpec((2, RB, DOUT), lambda i: (0, i, 0)),
                  pl.BlockSpec((2, RB, 32), lambda i: (0, i, 0)),
                  pl.BlockSpec((RB, 16), lambda i: (i, 0)),
                  pl.BlockSpec((RB, DOUT), lambda i: (i, 0)),
                  pl.BlockSpec((DOUT,), lambda i: (0,))],
        out_specs=pl.BlockSpec((RB, DOUT), lambda i: (i, 0)),
        out_shape=jax.ShapeDtypeStruct((N, DOUT), jnp.float32),
    )(accg, den_raw, aux, h2, b_gat)


def kernel(x, edge_index, W_gcn, b_gcn, W_gat, att_src, att_dst, b_gat):
    src = edge_index[0]
    dst = edge_index[1]
    pad = EP - E
    src_p = jnp.concatenate(
        [src, jnp.zeros((pad,), jnp.int32)]).reshape(IDX_ROWS, 128)
    dst_p = jnp.concatenate(
        [dst, jnp.full((pad,), TRASH, jnp.int32)]).reshape(IDX_ROWS, 128)

    deg_raw = _sc_deg(dst_p)
    h = _tc_h(x, W_gcn)
    gsplit2 = _tc_scale(deg_raw, h)
    agg = _sc_gcn(gsplit2.reshape(2 * N, 128), src_p.reshape(5120, 64),
                  dst_p.reshape(5120, 64))
    h2 = _tc_gcn_finish(agg, gsplit2, deg_raw, b_gcn, W_gat)
    a_s, a_d, fs16, aux, ca16 = _tc_att(h2, att_src, att_dst)
    gidx, dstc, den_part = _sc_class(src_p, dst_p, a_s, a_d, ca16)
    t2 = _tc_build_t(h2, fs16)
    accg = _sc_gat(t2.reshape(2 * N, 128), gidx, dstc)
    den_raw = jnp.swapaxes(den_part, 1, 2)
    return _tc_final(accg, den_raw, aux, h2, b_gat)
